# Initial kernel scaffold; baseline (speedup 1.0000x reference)
import jax
import jax.numpy as jnp
from jax.experimental import pallas as pl


def kernel(inputs, embedding):
    raise NotImplementedError("write your pallas kernel here")


# speedup vs baseline: 1.2958x; 1.2958x over previous
import functools

import jax
import jax.numpy as jnp
from jax import lax
from jax.experimental import pallas as pl
from jax.experimental.pallas import tpu as pltpu
from jax.experimental.pallas import tpu_sc as plsc

NUM_EMBEDDINGS = 1000000
EMBED_DIM = 32
BATCH = 16384
HIST = 50

LANE = 128
TOTAL = BATCH * HIST
NROWS = TOTAL // LANE
NW = 32
ROWS_PER_W = NROWS // NW
KK = 8
NGRP = ROWS_PER_W // KK


def _build_kernel():
    mesh = plsc.VectorSubcoreMesh(core_axis_name="c", subcore_axis_name="s")

    @functools.partial(
        pl.kernel,
        mesh=mesh,
        out_type=jax.ShapeDtypeStruct((NROWS, LANE, EMBED_DIM), jnp.float32),
        scratch_types=[
            pltpu.VMEM((ROWS_PER_W, LANE), jnp.int32),
            pltpu.VMEM((KK, LANE, EMBED_DIM), jnp.float32),
            pltpu.SemaphoreType.DMA,
        ],
        compiler_params=pltpu.CompilerParams(use_tc_tiling_on_sc=False),
    )
    def gather_kernel(idx_hbm, table_hbm, out_hbm, idx_v, rows_v, sem):
        wid = lax.axis_index("s") * 2 + lax.axis_index("c")
        base = wid * ROWS_PER_W
        pltpu.sync_copy(idx_hbm.at[pl.ds(base, ROWS_PER_W)], idx_v)

        def group(g, carry):
            copies = [
                pltpu.async_copy(
                    table_hbm.at[idx_v.at[g * KK + j]], rows_v.at[j], sem
                )
                for j in range(KK)
            ]
            for c in copies:
                c.wait()
            pltpu.sync_copy(rows_v, out_hbm.at[pl.ds(base + g * KK, KK)])
            return carry

        lax.fori_loop(0, NGRP, group, 0)

    return gather_kernel


_gather = _build_kernel()


@jax.jit
def kernel(inputs, embedding):
    idx = inputs.astype(jnp.int32).reshape(NROWS, LANE)
    out = _gather(idx, embedding)
    return out.reshape(BATCH, HIST, EMBED_DIM)


# speedup vs baseline: 1.3116x; 1.0122x over previous
import functools

import jax
import jax.numpy as jnp
from jax import lax
from jax.experimental import pallas as pl
from jax.experimental.pallas import tpu as pltpu
from jax.experimental.pallas import tpu_sc as plsc

NUM_EMBEDDINGS = 1000000
EMBED_DIM = 32
BATCH = 16384
HIST = 50

LANE = 128
TOTAL = BATCH * HIST
NROWS = TOTAL // LANE
NW = 32
ROWS_PER_W = NROWS // NW
KK = 10
NGRP = ROWS_PER_W // KK


def _build_kernel():
    mesh = plsc.VectorSubcoreMesh(core_axis_name="c", subcore_axis_name="s")

    @functools.partial(
        pl.kernel,
        mesh=mesh,
        out_type=jax.ShapeDtypeStruct((NROWS, LANE, EMBED_DIM), jnp.float32),
        scratch_types=[
            pltpu.VMEM((ROWS_PER_W, LANE), jnp.int32),
            pltpu.VMEM((2, KK, LANE, EMBED_DIM), jnp.float32),
            pltpu.SemaphoreType.DMA,
            pltpu.SemaphoreType.DMA,
        ],
        compiler_params=pltpu.CompilerParams(use_tc_tiling_on_sc=False),
    )
    def gather_kernel(idx_hbm, table_hbm, out_hbm, idx_v, rows_v, gsem, osem):
        wid = lax.axis_index("s") * 2 + lax.axis_index("c")
        base = wid * ROWS_PER_W
        pltpu.sync_copy(idx_hbm.at[pl.ds(base, ROWS_PER_W)], idx_v)

        def fire(g, b):
            for j in range(KK):
                pltpu.async_copy(
                    table_hbm.at[idx_v.at[g * KK + j]], rows_v.at[b, j], gsem
                )

        def drain_gathers(b):
            for j in range(KK):
                pltpu.make_async_copy(
                    table_hbm.at[idx_v.at[j]], rows_v.at[b, j], gsem
                ).wait()

        def wait_outcopy(b, g):
            pltpu.make_async_copy(
                rows_v.at[b], out_hbm.at[pl.ds(base + g * KK, KK)], osem
            ).wait()

        fire(0, 0)

        def step(g, carry):
            b_cur = lax.rem(g, 2)
            b_nxt = 1 - b_cur

            @pl.when(g > 0)
            def _():
                wait_outcopy(b_nxt, g - 1)

            fire(g + 1, b_nxt)
            drain_gathers(b_cur)
            pltpu.async_copy(
                rows_v.at[b_cur], out_hbm.at[pl.ds(base + g * KK, KK)], osem
            )
            return carry

        lax.fori_loop(0, NGRP - 1, step, 0)

        b_last = (NGRP - 1) % 2
        wait_outcopy(1 - b_last, NGRP - 2)
        drain_gathers(b_last)
        pltpu.sync_copy(
            rows_v.at[b_last], out_hbm.at[pl.ds(base + (NGRP - 1) * KK, KK)]
        )

    return gather_kernel


_gather = _build_kernel()


@jax.jit
def kernel(inputs, embedding):
    idx = inputs.astype(jnp.int32).reshape(NROWS, LANE)
    out = _gather(idx, embedding)
    return out.reshape(BATCH, HIST, EMBED_DIM)
